# baseline (speedup 1.0000x reference)
import jax
import jax.numpy as jnp
from jax.experimental import pallas as pl

_B = 2
_N = 2048
_NS = 1000
_K = 64


def _fps_kernel(px_ref, py_ref, pz_ref, out_ref):
    px = px_ref[0]
    py = py_ref[0]
    pz = pz_ref[0]
    R, C = px.shape
    n = R * C
    iota = (jax.lax.broadcasted_iota(jnp.int32, (R, C), 0) * C
            + jax.lax.broadcasted_iota(jnp.int32, (R, C), 1))

    def coords(idx):
        m = iota == idx
        return (jnp.sum(jnp.where(m, px, 0.0)),
                jnp.sum(jnp.where(m, py, 0.0)),
                jnp.sum(jnp.where(m, pz, 0.0)))

    sx, sy, sz = coords(0)
    dx = px - sx
    dy = py - sy
    dz = pz - sz
    dist = (dx * dx + dy * dy) + dz * dz
    ord0 = jnp.zeros((R, C), jnp.int32)

    def body(i, st):
        dcur, o = st
        m = jnp.max(dcur)
        nxt = jnp.min(jnp.where(dcur == m, iota, n)).astype(jnp.int32)
        o = jnp.where(iota == i, nxt, o)
        zx, zy, zz = coords(nxt)
        ddx = px - zx
        ddy = py - zy
        ddz = pz - zz
        nd = (ddx * ddx + ddy * ddy) + ddz * ddz
        return (jnp.minimum(dcur, nd), o)

    dist, o = jax.lax.fori_loop(1, n, body, (dist, ord0))
    out_ref[0] = o


def _fps_pallas(posb):
    Bn, N, _ = posb.shape
    R, C = 8, N // 8
    px = posb[:, :, 0].reshape(Bn, R, C)
    py = posb[:, :, 1].reshape(Bn, R, C)
    pz = posb[:, :, 2].reshape(Bn, R, C)
    out = pl.pallas_call(
        _fps_kernel,
        grid=(Bn,),
        in_specs=[pl.BlockSpec((1, R, C), lambda b: (b, 0, 0))] * 3,
        out_specs=pl.BlockSpec((1, R, C), lambda b: (b, 0, 0)),
        out_shape=jax.ShapeDtypeStruct((Bn, R, C), jnp.int32),
    )(px, py, pz)
    return out.reshape(Bn, N)


def _linmlp(ps, x):
    for i, (w, b) in enumerate(ps):
        x = x @ w + b
        if i < len(ps) - 1:
            x = jax.nn.relu(x)
    return x


def _chunked_topk(neg, k, chunk=128):
    r, n = neg.shape
    nc = n // chunk
    v1, i1 = jax.lax.top_k(neg.reshape(r, nc, chunk), k)
    i1 = i1 + (jnp.arange(nc, dtype=jnp.int32) * chunk)[None, :, None]
    v2, p2 = jax.lax.top_k(v1.reshape(r, nc * k), k)
    idx = jnp.take_along_axis(i1.reshape(r, nc * k), p2, axis=1)
    return v2, idx


def _sa_body(x, pos, q, r, ps):
    d2 = jnp.sum((q[:, None, :] - pos[None, :, :]) ** 2, axis=-1)
    d2m = jnp.where(d2 <= r * r, d2, jnp.inf)
    negv, nbr = _chunked_topk(-d2m, _K)
    valid = negv > -jnp.inf
    msg = jnp.concatenate([x[nbr], pos[nbr] - q[:, None, :]], axis=-1)
    sl = jnp.concatenate([x, pos - q], axis=-1)[:, None, :]
    msg = jnp.concatenate([msg, sl], axis=1)
    valid = jnp.concatenate(
        [valid, jnp.ones((q.shape[0], 1), dtype=jnp.bool_)], axis=1)
    h = _linmlp(ps, msg)
    h = jnp.where(valid[..., None], h, -jnp.inf)
    return jnp.max(h, axis=1)


def _knn_interp(x_src, pos_src, pos_dst, k):
    d2 = jnp.sum((pos_dst[:, None, :] - pos_src[None, :, :]) ** 2, axis=-1)
    negd, idx = jax.lax.top_k(-d2, k)
    w = 1.0 / jnp.clip(-negd, 1e-16, None)
    return jnp.sum(w[..., None] * x_src[idx], axis=1) / jnp.sum(w, axis=1, keepdims=True)


def kernel(pos, y, approach_scores, batch, params):
    posb = pos.reshape(_B, _N, 3)

    perm1 = _fps_pallas(posb)
    q1 = jax.vmap(lambda p, i: p[i])(posb, perm1)
    x1 = jax.vmap(lambda xb, pb, qb: _sa_body(xb, pb, qb, 0.1, params['sa1']))(
        posb, posb, q1)
    pos1 = q1

    x2 = jax.vmap(lambda xb, pb, qb: _sa_body(xb, pb, qb, 0.2, params['sa2']))(
        x1, pos1, pos1)
    pos2 = pos1

    gh = _linmlp(params['sa3'], jnp.concatenate([x2, pos2], axis=-1))
    gx = jnp.max(gh, axis=1)
    gpos = jnp.zeros((_B, 1, 3), jnp.float32)
    f3 = jax.vmap(lambda xs, psrc, pd: _knn_interp(xs, psrc, pd, 1))(
        gx[:, None, :], gpos, pos2)
    f3 = _linmlp(params['fp3'], jnp.concatenate([f3, x2], axis=-1))
    f2 = jax.vmap(lambda xs, psrc, pd: _knn_interp(xs, psrc, pd, 3))(
        f3, pos2, pos1)
    f2 = _linmlp(params['fp2'], jnp.concatenate([f2, x1], axis=-1))
    f1 = jax.vmap(lambda xs, psrc, pd: _knn_interp(xs, psrc, pd, 3))(
        f2, pos1, posb)
    f1 = _linmlp(params['fp1'], jnp.concatenate([f1, posb], axis=-1))
    h = jax.nn.relu(f1 @ params['head'][0][0] + params['head'][0][1])
    a = jax.nn.sigmoid(h @ params['head'][1][0] + params['head'][1][1])[..., 0]

    yb = y.reshape(_B, _N, 16)
    key = jax.random.key(123)
    idx_all, gt_all, ap_all = [], [], []
    for b in range(_B):
        p = jax.lax.stop_gradient(a[b])
        p = p / jnp.sum(p)
        ib = jax.random.choice(jax.random.fold_in(key, b), _N, shape=(_NS,),
                               replace=False, p=p)
        idx_all.append(ib + b * _N)
        gt_all.append(yb[b][ib])
        ap_all.append(posb[b][ib])
    gidx = jnp.concatenate(idx_all, axis=0)
    grasp_gt = jnp.stack(gt_all, axis=0).reshape(-1, 16)
    approach_points = jnp.stack(ap_all, axis=0).reshape(-1, 3)
    local = jnp.concatenate([x1.reshape(_B * _N, -1), x2.reshape(_B * _N, -1)],
                            axis=1)
    sel = local[gidx]
    rep_g = jnp.tile(gx, (_NS, 1))
    feats = jnp.concatenate([sel, rep_g], axis=1)
    af = jax.nn.relu(approach_points @ params['ae'][0][0] + params['ae'][0][1])
    af = af @ params['ae'][1][0] + params['ae'][1][1]
    xg = jnp.concatenate([feats, af], axis=1)
    for i, (w, bb) in enumerate(params['gp']):
        xg = xg @ w + bb
        if i < len(params['gp']) - 1:
            xg = jax.nn.relu(xg)
    grasp_pred = xg
    grasp_loss = jnp.mean((grasp_pred - grasp_gt) ** 2)
    agt = (approach_scores > 0).astype(jnp.float32).reshape(_B, _N)
    pcl = jnp.clip(a, 1e-7, 1.0 - 1e-7)
    bce = -(agt * jnp.log(pcl) + (1.0 - agt) * jnp.log(1.0 - pcl))
    approach_loss = jnp.mean(jnp.mean(bce, axis=1))
    return (grasp_pred, a, grasp_gt, grasp_loss, approach_loss, approach_points)


# speedup vs baseline: 1.0205x; 1.0205x over previous
import jax
import jax.numpy as jnp
from jax.experimental import pallas as pl
from jax.experimental.pallas import tpu as pltpu

_B = 2
_N = 2048
_NS = 1000
_K = 64


def _fps_kernel(cv_ref, cs_ref, out_ref):
    R, C = 8, 256
    n = R * C
    iota = (jax.lax.broadcasted_iota(jnp.int32, (R, C), 0) * C
            + jax.lax.broadcasted_iota(jnp.int32, (R, C), 1))

    coords = []
    dists = []
    for b in range(2):
        px = cv_ref[b, 0]
        py = cv_ref[b, 1]
        pz = cv_ref[b, 2]
        out_ref[b, 0] = 0
        sx = cs_ref[3 * b + 0, 0]
        sy = cs_ref[3 * b + 1, 0]
        sz = cs_ref[3 * b + 2, 0]
        dx = px - sx
        dy = py - sy
        dz = pz - sz
        coords.append((px, py, pz))
        dists.append((dx * dx + dy * dy) + dz * dz)

    def body(i, st):
        new = []
        for b in range(2):
            d = st[b]
            px, py, pz = coords[b]
            m = jnp.max(d)
            nxt = jnp.min(jnp.where(d == m, iota, n)).astype(jnp.int32)
            out_ref[b, i] = nxt
            zx = cs_ref[3 * b + 0, nxt]
            zy = cs_ref[3 * b + 1, nxt]
            zz = cs_ref[3 * b + 2, nxt]
            ddx = px - zx
            ddy = py - zy
            ddz = pz - zz
            nd = (ddx * ddx + ddy * ddy) + ddz * ddz
            new.append(jnp.minimum(d, nd))
        return tuple(new)

    jax.lax.fori_loop(1, n, body, tuple(dists))


def _fps_pallas(posb):
    Bn, N, _ = posb.shape
    cv = posb.transpose(0, 2, 1).reshape(Bn, 3, 8, N // 8)
    cs = posb.transpose(0, 2, 1).reshape(Bn * 3, N)
    out = pl.pallas_call(
        _fps_kernel,
        in_specs=[
            pl.BlockSpec(memory_space=pltpu.VMEM),
            pl.BlockSpec(memory_space=pltpu.SMEM),
        ],
        out_specs=pl.BlockSpec(memory_space=pltpu.SMEM),
        out_shape=jax.ShapeDtypeStruct((Bn, N), jnp.int32),
    )(cv, cs)
    return out


def _linmlp(ps, x):
    for i, (w, b) in enumerate(ps):
        x = x @ w + b
        if i < len(ps) - 1:
            x = jax.nn.relu(x)
    return x


def _chunked_topk(neg, k, chunk=128):
    r, n = neg.shape
    nc = n // chunk
    v1, i1 = jax.lax.top_k(neg.reshape(r, nc, chunk), k)
    i1 = i1 + (jnp.arange(nc, dtype=jnp.int32) * chunk)[None, :, None]
    v2, p2 = jax.lax.top_k(v1.reshape(r, nc * k), k)
    idx = jnp.take_along_axis(i1.reshape(r, nc * k), p2, axis=1)
    return v2, idx


def _sa_body(x, pos, q, r, ps):
    d2 = jnp.sum((q[:, None, :] - pos[None, :, :]) ** 2, axis=-1)
    d2m = jnp.where(d2 <= r * r, d2, jnp.inf)
    negv, nbr = _chunked_topk(-d2m, _K)
    valid = negv > -jnp.inf
    msg = jnp.concatenate([x[nbr], pos[nbr] - q[:, None, :]], axis=-1)
    sl = jnp.concatenate([x, pos - q], axis=-1)[:, None, :]
    msg = jnp.concatenate([msg, sl], axis=1)
    valid = jnp.concatenate(
        [valid, jnp.ones((q.shape[0], 1), dtype=jnp.bool_)], axis=1)
    h = _linmlp(ps, msg)
    h = jnp.where(valid[..., None], h, -jnp.inf)
    return jnp.max(h, axis=1)


def _knn_interp(x_src, pos_src, pos_dst, k):
    d2 = jnp.sum((pos_dst[:, None, :] - pos_src[None, :, :]) ** 2, axis=-1)
    negd, idx = jax.lax.top_k(-d2, k)
    w = 1.0 / jnp.clip(-negd, 1e-16, None)
    return jnp.sum(w[..., None] * x_src[idx], axis=1) / jnp.sum(w, axis=1, keepdims=True)


def kernel(pos, y, approach_scores, batch, params):
    posb = pos.reshape(_B, _N, 3)

    perm1 = _fps_pallas(posb)
    q1 = jax.vmap(lambda p, i: p[i])(posb, perm1)
    x1 = jax.vmap(lambda xb, pb, qb: _sa_body(xb, pb, qb, 0.1, params['sa1']))(
        posb, posb, q1)
    pos1 = q1

    x2 = jax.vmap(lambda xb, pb, qb: _sa_body(xb, pb, qb, 0.2, params['sa2']))(
        x1, pos1, pos1)
    pos2 = pos1

    gh = _linmlp(params['sa3'], jnp.concatenate([x2, pos2], axis=-1))
    gx = jnp.max(gh, axis=1)
    gpos = jnp.zeros((_B, 1, 3), jnp.float32)
    f3 = jax.vmap(lambda xs, psrc, pd: _knn_interp(xs, psrc, pd, 1))(
        gx[:, None, :], gpos, pos2)
    f3 = _linmlp(params['fp3'], jnp.concatenate([f3, x2], axis=-1))
    f2 = jax.vmap(lambda xs, psrc, pd: _knn_interp(xs, psrc, pd, 3))(
        f3, pos2, pos1)
    f2 = _linmlp(params['fp2'], jnp.concatenate([f2, x1], axis=-1))
    f1 = jax.vmap(lambda xs, psrc, pd: _knn_interp(xs, psrc, pd, 3))(
        f2, pos1, posb)
    f1 = _linmlp(params['fp1'], jnp.concatenate([f1, posb], axis=-1))
    h = jax.nn.relu(f1 @ params['head'][0][0] + params['head'][0][1])
    a = jax.nn.sigmoid(h @ params['head'][1][0] + params['head'][1][1])[..., 0]

    yb = y.reshape(_B, _N, 16)
    key = jax.random.key(123)
    idx_all, gt_all, ap_all = [], [], []
    for b in range(_B):
        p = jax.lax.stop_gradient(a[b])
        p = p / jnp.sum(p)
        ib = jax.random.choice(jax.random.fold_in(key, b), _N, shape=(_NS,),
                               replace=False, p=p)
        idx_all.append(ib + b * _N)
        gt_all.append(yb[b][ib])
        ap_all.append(posb[b][ib])
    gidx = jnp.concatenate(idx_all, axis=0)
    grasp_gt = jnp.stack(gt_all, axis=0).reshape(-1, 16)
    approach_points = jnp.stack(ap_all, axis=0).reshape(-1, 3)
    local = jnp.concatenate([x1.reshape(_B * _N, -1), x2.reshape(_B * _N, -1)],
                            axis=1)
    sel = local[gidx]
    rep_g = jnp.tile(gx, (_NS, 1))
    feats = jnp.concatenate([sel, rep_g], axis=1)
    af = jax.nn.relu(approach_points @ params['ae'][0][0] + params['ae'][0][1])
    af = af @ params['ae'][1][0] + params['ae'][1][1]
    xg = jnp.concatenate([feats, af], axis=1)
    for i, (w, bb) in enumerate(params['gp']):
        xg = xg @ w + bb
        if i < len(params['gp']) - 1:
            xg = jax.nn.relu(xg)
    grasp_pred = xg
    grasp_loss = jnp.mean((grasp_pred - grasp_gt) ** 2)
    agt = (approach_scores > 0).astype(jnp.float32).reshape(_B, _N)
    pcl = jnp.clip(a, 1e-7, 1.0 - 1e-7)
    bce = -(agt * jnp.log(pcl) + (1.0 - agt) * jnp.log(1.0 - pcl))
    approach_loss = jnp.mean(jnp.mean(bce, axis=1))
    return (grasp_pred, a, grasp_gt, grasp_loss, approach_loss, approach_points)


# speedup vs baseline: 2.2713x; 2.2257x over previous
import jax
import jax.numpy as jnp
from jax.experimental import pallas as pl
from jax.experimental.pallas import tpu as pltpu

_B = 2
_N = 2048
_NS = 1000
_K = 64


def _fps_kernel(cv_ref, cs_ref, out_ref):
    R, C = 8, 256
    n = R * C
    iota = (jax.lax.broadcasted_iota(jnp.int32, (R, C), 0) * C
            + jax.lax.broadcasted_iota(jnp.int32, (R, C), 1))

    coords = []
    dists = []
    for b in range(2):
        px = cv_ref[b, 0]
        py = cv_ref[b, 1]
        pz = cv_ref[b, 2]
        out_ref[b, 0] = 0
        sx = cs_ref[3 * b + 0, 0]
        sy = cs_ref[3 * b + 1, 0]
        sz = cs_ref[3 * b + 2, 0]
        dx = px - sx
        dy = py - sy
        dz = pz - sz
        coords.append((px, py, pz))
        dists.append((dx * dx + dy * dy) + dz * dz)

    def body(i, st):
        new = []
        for b in range(2):
            d = st[b]
            px, py, pz = coords[b]
            m = jnp.max(d)
            nxt = jnp.min(jnp.where(d == m, iota, n)).astype(jnp.int32)
            out_ref[b, i] = nxt
            zx = cs_ref[3 * b + 0, nxt]
            zy = cs_ref[3 * b + 1, nxt]
            zz = cs_ref[3 * b + 2, nxt]
            ddx = px - zx
            ddy = py - zy
            ddz = pz - zz
            nd = (ddx * ddx + ddy * ddy) + ddz * ddz
            new.append(jnp.minimum(d, nd))
        return tuple(new)

    jax.lax.fori_loop(1, n, body, tuple(dists))


def _fps_pallas(posb):
    Bn, N, _ = posb.shape
    cv = posb.transpose(0, 2, 1).reshape(Bn, 3, 8, N // 8)
    cs = posb.transpose(0, 2, 1).reshape(Bn * 3, N)
    out = pl.pallas_call(
        _fps_kernel,
        in_specs=[
            pl.BlockSpec(memory_space=pltpu.VMEM),
            pl.BlockSpec(memory_space=pltpu.SMEM),
        ],
        out_specs=pl.BlockSpec(memory_space=pltpu.SMEM),
        out_shape=jax.ShapeDtypeStruct((Bn, N), jnp.int32),
    )(cv, cs)
    return out


def _linmlp(ps, x):
    for i, (w, b) in enumerate(ps):
        x = x @ w + b
        if i < len(ps) - 1:
            x = jax.nn.relu(x)
    return x


def _chunked_topk(neg, k, chunk=128):
    r, n = neg.shape
    nc = n // chunk
    v1, i1 = jax.lax.top_k(neg.reshape(r, nc, chunk), k)
    i1 = i1 + (jnp.arange(nc, dtype=jnp.int32) * chunk)[None, :, None]
    v2, p2 = jax.lax.top_k(v1.reshape(r, nc * k), k)
    idx = jnp.take_along_axis(i1.reshape(r, nc * k), p2, axis=1)
    return v2, idx


def _sa_body(x, pos, q, r, ps):
    d2 = jnp.sum((q[:, None, :] - pos[None, :, :]) ** 2, axis=-1)
    d2m = jnp.where(d2 <= r * r, d2, jnp.inf)
    negv, nbr = jax.lax.top_k(-d2m, _K)
    valid = negv > -jnp.inf
    msg = jnp.concatenate([x[nbr], pos[nbr] - q[:, None, :]], axis=-1)
    sl = jnp.concatenate([x, pos - q], axis=-1)[:, None, :]
    msg = jnp.concatenate([msg, sl], axis=1)
    valid = jnp.concatenate(
        [valid, jnp.ones((q.shape[0], 1), dtype=jnp.bool_)], axis=1)
    h = _linmlp(ps, msg)
    h = jnp.where(valid[..., None], h, -jnp.inf)
    return jnp.max(h, axis=1)


def _knn_interp(x_src, pos_src, pos_dst, k):
    d2 = jnp.sum((pos_dst[:, None, :] - pos_src[None, :, :]) ** 2, axis=-1)
    cur = -d2
    ns = pos_src.shape[0]
    col = jnp.arange(ns, dtype=jnp.int32)[None, :]
    vs, ids = [], []
    for _ in range(k):
        v = jnp.max(cur, axis=-1)
        i = jnp.argmax(cur, axis=-1).astype(jnp.int32)
        vs.append(v)
        ids.append(i)
        if len(vs) < k:
            cur = jnp.where(col == i[:, None], -jnp.inf, cur)
    negd = jnp.stack(vs, axis=-1)
    idx = jnp.stack(ids, axis=-1)
    w = 1.0 / jnp.clip(-negd, 1e-16, None)
    return jnp.sum(w[..., None] * x_src[idx], axis=1) / jnp.sum(w, axis=1, keepdims=True)


def kernel(pos, y, approach_scores, batch, params):
    posb = pos.reshape(_B, _N, 3)

    perm1 = _fps_pallas(posb)
    q1 = jax.vmap(lambda p, i: p[i])(posb, perm1)
    x1 = jax.vmap(lambda xb, pb, qb: _sa_body(xb, pb, qb, 0.1, params['sa1']))(
        posb, posb, q1)
    pos1 = q1

    x2 = jax.vmap(lambda xb, pb, qb: _sa_body(xb, pb, qb, 0.2, params['sa2']))(
        x1, pos1, pos1)
    pos2 = pos1

    gh = _linmlp(params['sa3'], jnp.concatenate([x2, pos2], axis=-1))
    gx = jnp.max(gh, axis=1)
    gpos = jnp.zeros((_B, 1, 3), jnp.float32)
    f3 = jax.vmap(lambda xs, psrc, pd: _knn_interp(xs, psrc, pd, 1))(
        gx[:, None, :], gpos, pos2)
    f3 = _linmlp(params['fp3'], jnp.concatenate([f3, x2], axis=-1))
    f2 = jax.vmap(lambda xs, psrc, pd: _knn_interp(xs, psrc, pd, 3))(
        f3, pos2, pos1)
    f2 = _linmlp(params['fp2'], jnp.concatenate([f2, x1], axis=-1))
    f1 = jax.vmap(lambda xs, psrc, pd: _knn_interp(xs, psrc, pd, 3))(
        f2, pos1, posb)
    f1 = _linmlp(params['fp1'], jnp.concatenate([f1, posb], axis=-1))
    h = jax.nn.relu(f1 @ params['head'][0][0] + params['head'][0][1])
    a = jax.nn.sigmoid(h @ params['head'][1][0] + params['head'][1][1])[..., 0]

    yb = y.reshape(_B, _N, 16)
    key = jax.random.key(123)
    idx_all, gt_all, ap_all = [], [], []
    for b in range(_B):
        p = jax.lax.stop_gradient(a[b])
        p = p / jnp.sum(p)
        ib = jax.random.choice(jax.random.fold_in(key, b), _N, shape=(_NS,),
                               replace=False, p=p)
        idx_all.append(ib + b * _N)
        gt_all.append(yb[b][ib])
        ap_all.append(posb[b][ib])
    gidx = jnp.concatenate(idx_all, axis=0)
    grasp_gt = jnp.stack(gt_all, axis=0).reshape(-1, 16)
    approach_points = jnp.stack(ap_all, axis=0).reshape(-1, 3)
    local = jnp.concatenate([x1.reshape(_B * _N, -1), x2.reshape(_B * _N, -1)],
                            axis=1)
    sel = local[gidx]
    rep_g = jnp.tile(gx, (_NS, 1))
    feats = jnp.concatenate([sel, rep_g], axis=1)
    af = jax.nn.relu(approach_points @ params['ae'][0][0] + params['ae'][0][1])
    af = af @ params['ae'][1][0] + params['ae'][1][1]
    xg = jnp.concatenate([feats, af], axis=1)
    for i, (w, bb) in enumerate(params['gp']):
        xg = xg @ w + bb
        if i < len(params['gp']) - 1:
            xg = jax.nn.relu(xg)
    grasp_pred = xg
    grasp_loss = jnp.mean((grasp_pred - grasp_gt) ** 2)
    agt = (approach_scores > 0).astype(jnp.float32).reshape(_B, _N)
    pcl = jnp.clip(a, 1e-7, 1.0 - 1e-7)
    bce = -(agt * jnp.log(pcl) + (1.0 - agt) * jnp.log(1.0 - pcl))
    approach_loss = jnp.mean(jnp.mean(bce, axis=1))
    return (grasp_pred, a, grasp_gt, grasp_loss, approach_loss, approach_points)


# speedup vs baseline: 2.4753x; 1.0898x over previous
import jax
import jax.numpy as jnp
from jax.experimental import pallas as pl
from jax.experimental.pallas import tpu as pltpu

_B = 2
_N = 2048
_NS = 1000
_K = 64


def _fps_kernel(cv_ref, cs_ref, out_ref):
    R, C = 8, 256
    n = R * C
    iota = (jax.lax.broadcasted_iota(jnp.int32, (R, C), 0) * C
            + jax.lax.broadcasted_iota(jnp.int32, (R, C), 1))

    coords = []
    dists = []
    for b in range(2):
        px = cv_ref[b, 0]
        py = cv_ref[b, 1]
        pz = cv_ref[b, 2]
        out_ref[b, 0] = 0
        sx = cs_ref[3 * b + 0, 0]
        sy = cs_ref[3 * b + 1, 0]
        sz = cs_ref[3 * b + 2, 0]
        dx = px - sx
        dy = py - sy
        dz = pz - sz
        coords.append((px, py, pz))
        dists.append((dx * dx + dy * dy) + dz * dz)

    def body(i, st):
        new = []
        for b in range(2):
            d = st[b]
            px, py, pz = coords[b]
            m = jnp.max(d)
            nxt = jnp.min(jnp.where(d == m, iota, n)).astype(jnp.int32)
            out_ref[b, i] = nxt
            zx = cs_ref[3 * b + 0, nxt]
            zy = cs_ref[3 * b + 1, nxt]
            zz = cs_ref[3 * b + 2, nxt]
            ddx = px - zx
            ddy = py - zy
            ddz = pz - zz
            nd = (ddx * ddx + ddy * ddy) + ddz * ddz
            new.append(jnp.minimum(d, nd))
        return tuple(new)

    jax.lax.fori_loop(1, n, body, tuple(dists))


def _fps_pallas(posb):
    Bn, N, _ = posb.shape
    cv = posb.transpose(0, 2, 1).reshape(Bn, 3, 8, N // 8)
    cs = posb.transpose(0, 2, 1).reshape(Bn * 3, N)
    out = pl.pallas_call(
        _fps_kernel,
        in_specs=[
            pl.BlockSpec(memory_space=pltpu.VMEM),
            pl.BlockSpec(memory_space=pltpu.SMEM),
        ],
        out_specs=pl.BlockSpec(memory_space=pltpu.SMEM),
        out_shape=jax.ShapeDtypeStruct((Bn, N), jnp.int32),
    )(cv, cs)
    return out


def _linmlp(ps, x):
    for i, (w, b) in enumerate(ps):
        x = x @ w + b
        if i < len(ps) - 1:
            x = jax.nn.relu(x)
    return x


def _chunked_topk(neg, k, chunk=128):
    r, n = neg.shape
    nc = n // chunk
    v1, i1 = jax.lax.top_k(neg.reshape(r, nc, chunk), k)
    i1 = i1 + (jnp.arange(nc, dtype=jnp.int32) * chunk)[None, :, None]
    v2, p2 = jax.lax.top_k(v1.reshape(r, nc * k), k)
    idx = jnp.take_along_axis(i1.reshape(r, nc * k), p2, axis=1)
    return v2, idx


def _topk64_set(d2m):
    bits = jax.lax.bitcast_convert_type(d2m, jnp.int32)
    rws, n = d2m.shape

    def bs_body(_, st):
        lo, hi = st
        mid = jax.lax.div(lo + hi, 2)
        c = jnp.sum((bits <= mid[:, None]).astype(jnp.int32), axis=1)
        ge = c >= _K
        return (jnp.where(ge, lo, mid + 1), jnp.where(ge, mid, hi))

    lo0 = jnp.zeros((rws,), jnp.int32)
    hi0 = jnp.full((rws,), 0x7F800000, jnp.int32)
    _, t = jax.lax.fori_loop(0, 31, bs_body, (lo0, hi0))

    lt = bits < t[:, None]
    c1 = jnp.sum(lt.astype(jnp.int32), axis=1)
    eq = bits == t[:, None]
    tier = jnp.cumsum(eq.astype(jnp.int32), axis=1)
    sel = lt | (eq & (tier <= (_K - c1)[:, None]))
    cum = jnp.cumsum(sel.astype(jnp.int32), axis=1)
    s = jnp.arange(_K, dtype=jnp.int32)
    nbr = jnp.sum((cum[:, :, None] <= s[None, None, :]).astype(jnp.int32), axis=1)
    negv = -jnp.take_along_axis(d2m, nbr, axis=1)
    return negv, nbr


def _sa_body(x, pos, q, r, ps):
    d2 = jnp.sum((q[:, None, :] - pos[None, :, :]) ** 2, axis=-1)
    d2m = jnp.where(d2 <= r * r, d2, jnp.inf)
    negv, nbr = _topk64_set(d2m)
    valid = negv > -jnp.inf
    msg = jnp.concatenate([x[nbr], pos[nbr] - q[:, None, :]], axis=-1)
    sl = jnp.concatenate([x, pos - q], axis=-1)[:, None, :]
    msg = jnp.concatenate([msg, sl], axis=1)
    valid = jnp.concatenate(
        [valid, jnp.ones((q.shape[0], 1), dtype=jnp.bool_)], axis=1)
    h = _linmlp(ps, msg)
    h = jnp.where(valid[..., None], h, -jnp.inf)
    return jnp.max(h, axis=1)


def _knn_interp(x_src, pos_src, pos_dst, k):
    d2 = jnp.sum((pos_dst[:, None, :] - pos_src[None, :, :]) ** 2, axis=-1)
    cur = -d2
    ns = pos_src.shape[0]
    col = jnp.arange(ns, dtype=jnp.int32)[None, :]
    vs, ids = [], []
    for _ in range(k):
        v = jnp.max(cur, axis=-1)
        i = jnp.argmax(cur, axis=-1).astype(jnp.int32)
        vs.append(v)
        ids.append(i)
        if len(vs) < k:
            cur = jnp.where(col == i[:, None], -jnp.inf, cur)
    negd = jnp.stack(vs, axis=-1)
    idx = jnp.stack(ids, axis=-1)
    w = 1.0 / jnp.clip(-negd, 1e-16, None)
    return jnp.sum(w[..., None] * x_src[idx], axis=1) / jnp.sum(w, axis=1, keepdims=True)


def kernel(pos, y, approach_scores, batch, params):
    posb = pos.reshape(_B, _N, 3)

    perm1 = _fps_pallas(posb)
    q1 = jax.vmap(lambda p, i: p[i])(posb, perm1)
    x1 = jax.vmap(lambda xb, pb, qb: _sa_body(xb, pb, qb, 0.1, params['sa1']))(
        posb, posb, q1)
    pos1 = q1

    x2 = jax.vmap(lambda xb, pb, qb: _sa_body(xb, pb, qb, 0.2, params['sa2']))(
        x1, pos1, pos1)
    pos2 = pos1

    gh = _linmlp(params['sa3'], jnp.concatenate([x2, pos2], axis=-1))
    gx = jnp.max(gh, axis=1)
    gpos = jnp.zeros((_B, 1, 3), jnp.float32)
    f3 = jax.vmap(lambda xs, psrc, pd: _knn_interp(xs, psrc, pd, 1))(
        gx[:, None, :], gpos, pos2)
    f3 = _linmlp(params['fp3'], jnp.concatenate([f3, x2], axis=-1))
    f2 = jax.vmap(lambda xs, psrc, pd: _knn_interp(xs, psrc, pd, 3))(
        f3, pos2, pos1)
    f2 = _linmlp(params['fp2'], jnp.concatenate([f2, x1], axis=-1))
    f1 = jax.vmap(lambda xs, psrc, pd: _knn_interp(xs, psrc, pd, 3))(
        f2, pos1, posb)
    f1 = _linmlp(params['fp1'], jnp.concatenate([f1, posb], axis=-1))
    h = jax.nn.relu(f1 @ params['head'][0][0] + params['head'][0][1])
    a = jax.nn.sigmoid(h @ params['head'][1][0] + params['head'][1][1])[..., 0]

    yb = y.reshape(_B, _N, 16)
    key = jax.random.key(123)
    idx_all, gt_all, ap_all = [], [], []
    for b in range(_B):
        p = jax.lax.stop_gradient(a[b])
        p = p / jnp.sum(p)
        ib = jax.random.choice(jax.random.fold_in(key, b), _N, shape=(_NS,),
                               replace=False, p=p)
        idx_all.append(ib + b * _N)
        gt_all.append(yb[b][ib])
        ap_all.append(posb[b][ib])
    gidx = jnp.concatenate(idx_all, axis=0)
    grasp_gt = jnp.stack(gt_all, axis=0).reshape(-1, 16)
    approach_points = jnp.stack(ap_all, axis=0).reshape(-1, 3)
    local = jnp.concatenate([x1.reshape(_B * _N, -1), x2.reshape(_B * _N, -1)],
                            axis=1)
    sel = local[gidx]
    rep_g = jnp.tile(gx, (_NS, 1))
    feats = jnp.concatenate([sel, rep_g], axis=1)
    af = jax.nn.relu(approach_points @ params['ae'][0][0] + params['ae'][0][1])
    af = af @ params['ae'][1][0] + params['ae'][1][1]
    xg = jnp.concatenate([feats, af], axis=1)
    for i, (w, bb) in enumerate(params['gp']):
        xg = xg @ w + bb
        if i < len(params['gp']) - 1:
            xg = jax.nn.relu(xg)
    grasp_pred = xg
    grasp_loss = jnp.mean((grasp_pred - grasp_gt) ** 2)
    agt = (approach_scores > 0).astype(jnp.float32).reshape(_B, _N)
    pcl = jnp.clip(a, 1e-7, 1.0 - 1e-7)
    bce = -(agt * jnp.log(pcl) + (1.0 - agt) * jnp.log(1.0 - pcl))
    approach_loss = jnp.mean(jnp.mean(bce, axis=1))
    return (grasp_pred, a, grasp_gt, grasp_loss, approach_loss, approach_points)
